# Initial kernel scaffold; baseline (speedup 1.0000x reference)
import jax
import jax.numpy as jnp
from jax.experimental import pallas as pl


def kernel(batch, enc_W1, enc_b1, enc_W2, enc_b2, enc_W3, enc_b3, g1_Wl, g1_bl, g1_Wr, g1_br, g1_We, g1_att, g1_bias, g2_Wl, g2_bl, g2_Wr, g2_br, g2_We, g2_att, g2_bias, g3_Wl, g3_bl, g3_Wr, g3_br, g3_We, g3_att, g3_bias, skip_W1, skip_b1, skip_W2, skip_b2, skip_W3, skip_b3, head_W1, head_b1, head_W2, head_b2, head_W3, head_b3):
    raise NotImplementedError("write your pallas kernel here")


# speedup vs baseline: 1.1518x; 1.1518x over previous
import functools

import jax
import jax.numpy as jnp
from jax.experimental import pallas as pl
from jax.experimental.pallas import tpu as pltpu

N = 256
BIG = 1e30
NEG = -1e9


def _mlp3(x, W1, b1, W2, b2, W3, b3):
    h = jnp.maximum(jnp.dot(x, W1, preferred_element_type=jnp.float32) + b1, 0.0)
    h = jnp.maximum(jnp.dot(h, W2, preferred_element_type=jnp.float32) + b2, 0.0)
    return jnp.dot(h, W3, preferred_element_type=jnp.float32) + b3


def _leaky(x):
    return jnp.where(x >= 0, x, 0.2 * x)


def _gat_layer(x, maskf_ref, D_ref, Wl, bl, Wr, br, We_col, att_col, bias,
               c_in, alpha_ref, xr_ref):
    if c_in == 1:
        xl = x * Wl + bl
        xr = x * Wr + br
    else:
        xl = jnp.dot(x, Wl, preferred_element_type=jnp.float32) + bl
        xr = jnp.dot(x, Wr, preferred_element_type=jnp.float32) + br
    xlT = xl.T
    c = xr.shape[1]
    xr_ref[:, :c] = xr

    def body(ib, _):
        base = ib * 8
        xr_blk = xr_ref[pl.ds(base, 8), :c]
        xrT_blk = xr_blk.T
        D_blk = D_ref[pl.ds(base, 8), :]
        m_blk = maskf_ref[pl.ds(base, 8), :]
        rows = []
        for t in range(8):
            xr_col = xrT_blk[:, t:t + 1]
            d_row = D_blk[t:t + 1, :]
            h = _leaky(xr_col + xlT + d_row * We_col)
            lg = jnp.sum(h * att_col, axis=0, keepdims=True)
            lg = jnp.where(m_blk[t:t + 1, :] > 0, lg, NEG)
            mx = jnp.max(lg, axis=1, keepdims=True)
            e = jnp.exp(lg - mx)
            rows.append(e / jnp.sum(e, axis=1, keepdims=True))
        alpha_ref[pl.ds(base, 8), :] = jnp.concatenate(rows, axis=0)
        return 0

    jax.lax.fori_loop(0, N // 8, body, 0)
    out = jnp.dot(alpha_ref[...], xl, preferred_element_type=jnp.float32)
    return out + bias


def _fwd_kernel(batch_ref,
                enc_W1, enc_b1, enc_W2, enc_b2, enc_W3, enc_b3,
                g1_Wl, g1_bl, g1_Wr, g1_br, g1_We, g1_att, g1_bias,
                g2_Wl, g2_bl, g2_Wr, g2_br, g2_We, g2_att, g2_bias,
                g3_Wl, g3_bl, g3_Wr, g3_br, g3_We, g3_att, g3_bias,
                skip_W1, skip_b1, skip_W2, skip_b2, skip_W3, skip_b3,
                head_W1, head_b1, head_W2, head_b2, head_W3, head_b3,
                rec_ref, lat_ref,
                M_ref, D_ref, maskf_ref, alpha_ref, AT_ref, xr_ref):
    x_in = batch_ref[0]
    latent = _mlp3(x_in, enc_W1[...], enc_b1[...], enc_W2[...], enc_b2[...],
                   enc_W3[...], enc_b3[...])
    lat_ref[0] = latent

    cols = [latent[:, c:c + 1] for c in range(3)]
    G = cols[0] * cols[0].T + cols[1] * cols[1].T + cols[2] * cols[2].T
    s = jnp.sum(latent * latent, axis=1, keepdims=True)
    eye = jnp.eye(N, dtype=jnp.float32)
    AT = s.T - G
    AT_ref[...] = AT + eye * BIG
    G2 = G - eye * BIG

    def mp_body(ib, _):
        base = ib * 8
        at_blk = AT_ref[pl.ds(base, 8), :]
        at_t = at_blk.T
        rows = []
        for t in range(8):
            u = at_t[:, t:t + 1]
            v = u - G2
            rows.append(jnp.min(v, axis=0, keepdims=True))
        M_ref[pl.ds(base, 8), :] = jnp.concatenate(rows, axis=0)
        return 0

    jax.lax.fori_loop(0, N // 8, mp_body, 0)
    maskf = jnp.where((M_ref[...] + G >= 0) & (eye == 0), 1.0, 0.0)
    maskf_ref[...] = maskf

    d2f = jnp.zeros((N, N), dtype=jnp.float32)
    for c in range(3):
        col = latent[:, c:c + 1]
        diff = col - col.T
        d2f = d2f + diff * diff
    pos = d2f > 0.0
    dist = jnp.where(pos, jnp.sqrt(jnp.where(pos, d2f, 1.0)), 0.0)
    ecnt = jnp.sum(maskf)
    mean_attr = jnp.sum(maskf * dist) / jnp.maximum(ecnt, 1.0)
    D = jnp.where(eye > 0, mean_attr, dist)
    D_ref[...] = D
    maskf_full = maskf + eye
    maskf_ref[...] = maskf_full

    x0 = latent[:, 2:3]
    x1 = _gat_layer(x0, maskf_ref, D_ref, g1_Wl[...], g1_bl[...], g1_Wr[...],
                    g1_br[...], g1_We[...], g1_att[...], g1_bias[...], 1,
                    alpha_ref, xr_ref)
    x1 = jnp.maximum(x1, 0.0)
    x2 = _gat_layer(x1, maskf_ref, D_ref, g2_Wl[...], g2_bl[...], g2_Wr[...],
                    g2_br[...], g2_We[...], g2_att[...], g2_bias[...], 64,
                    alpha_ref, xr_ref)
    x2 = jnp.maximum(x2, 0.0)
    g = _gat_layer(x2, maskf_ref, D_ref, g3_Wl[...], g3_bl[...], g3_Wr[...],
                   g3_br[...], g3_We[...], g3_att[...], g3_bias[...], 64,
                   alpha_ref, xr_ref)

    comb = g + 0.1 * _mlp3(latent, skip_W1[...], skip_b1[...], skip_W2[...],
                           skip_b2[...], skip_W3[...], skip_b3[...])
    rec_ref[0] = _mlp3(comb, head_W1[...], head_b1[...], head_W2[...],
                       head_b2[...], head_W3[...], head_b3[...])


def kernel(batch, enc_W1, enc_b1, enc_W2, enc_b2, enc_W3, enc_b3,
           g1_Wl, g1_bl, g1_Wr, g1_br, g1_We, g1_att, g1_bias,
           g2_Wl, g2_bl, g2_Wr, g2_br, g2_We, g2_att, g2_bias,
           g3_Wl, g3_bl, g3_Wr, g3_br, g3_We, g3_att, g3_bias,
           skip_W1, skip_b1, skip_W2, skip_b2, skip_W3, skip_b3,
           head_W1, head_b1, head_W2, head_b2, head_W3, head_b3):
    B = batch.shape[0]
    f32 = jnp.float32

    row = lambda v: v.reshape(1, -1)
    col = lambda v: v.reshape(-1, 1)
    args = (
        batch,
        enc_W1, row(enc_b1), enc_W2, row(enc_b2), enc_W3, row(enc_b3),
        g1_Wl, row(g1_bl), g1_Wr, row(g1_br), col(g1_We), col(g1_att), row(g1_bias),
        g2_Wl, row(g2_bl), g2_Wr, row(g2_br), col(g2_We), col(g2_att), row(g2_bias),
        g3_Wl, row(g3_bl), g3_Wr, row(g3_br), col(g3_We), col(g3_att), row(g3_bias),
        skip_W1, row(skip_b1), skip_W2, row(skip_b2), skip_W3, row(skip_b3),
        head_W1, row(head_b1), head_W2, row(head_b2), head_W3, row(head_b3),
    )

    in_specs = [pl.BlockSpec((1, N, batch.shape[2]), lambda b: (b, 0, 0))]
    for a in args[1:]:
        in_specs.append(
            pl.BlockSpec(a.shape, functools.partial(lambda nd, b: (0,) * nd,
                                                    a.ndim)))

    out_shapes = (
        jax.ShapeDtypeStruct((B, N, 12), f32),
        jax.ShapeDtypeStruct((B, N, 3), f32),
    )
    out_specs = (
        pl.BlockSpec((1, N, 12), lambda b: (b, 0, 0)),
        pl.BlockSpec((1, N, 3), lambda b: (b, 0, 0)),
    )
    scratch = [
        pltpu.VMEM((N, N), f32),
        pltpu.VMEM((N, N), f32),
        pltpu.VMEM((N, N), f32),
        pltpu.VMEM((N, N), f32),
        pltpu.VMEM((N, N), f32),
        pltpu.VMEM((N, 64), f32),
    ]
    rec, lat = pl.pallas_call(
        _fwd_kernel,
        grid=(B,),
        in_specs=in_specs,
        out_specs=out_specs,
        out_shape=out_shapes,
        scratch_shapes=scratch,
        compiler_params=pltpu.CompilerParams(
            dimension_semantics=("parallel",)),
    )(*args)
    return (batch, rec, lat)
